# Initial kernel scaffold; baseline (speedup 1.0000x reference)
import functools

import jax
import jax.numpy as jnp
from jax import lax
from jax.experimental import pallas as pl
from jax.experimental.pallas import tpu as pltpu
from jax.experimental.pallas import tpu_sc as plsc


def _make_gather2(N, D, C, NC, NS):
    NW = NC * NS
    per_w = N // NW
    n_chunks = per_w // C
    mesh = plsc.VectorSubcoreMesh(core_axis_name="c", subcore_axis_name="s")

    @functools.partial(
        pl.kernel,
        mesh=mesh,
        out_type=jax.ShapeDtypeStruct((N, 2, D), jnp.float32),
        scratch_types=[
            pltpu.VMEM((C,), jnp.int32),
            pltpu.VMEM((C,), jnp.int32),
            pltpu.VMEM((C, 1, D), jnp.float32),
            pltpu.VMEM((C, 1, D), jnp.float32),
            pltpu.SemaphoreType.DMA,
        ],
    )
    def gather2(word_hbm, entity_hbm, wid_hbm, eid_hbm, out_hbm,
                widx_v, eidx_v, wrows_v, erows_v, sem):
        wid = lax.axis_index("s") * NC + lax.axis_index("c")
        base = wid * per_w

        def body(i, carry):
            start = base + i * C
            pltpu.sync_copy(wid_hbm.at[pl.ds(start, C)], widx_v)
            pltpu.sync_copy(eid_hbm.at[pl.ds(start, C)], eidx_v)
            cw = pltpu.async_copy(word_hbm.at[widx_v], wrows_v.at[:, 0], sem)
            ce = pltpu.async_copy(entity_hbm.at[eidx_v], erows_v.at[:, 0], sem)
            cw.wait()
            ce.wait()
            pltpu.sync_copy(wrows_v, out_hbm.at[pl.ds(start, C), pl.ds(0, 1)])
            pltpu.sync_copy(erows_v, out_hbm.at[pl.ds(start, C), pl.ds(1, 1)])
            return carry

        lax.fori_loop(0, n_chunks, body, 0, unroll=False)

    return gather2


def kernel(word_table, entity_table, word_ids, entity_ids):
    B, H = word_ids.shape
    D = word_table.shape[1]
    N = B * H
    info = plsc.get_sparse_core_info()
    NC, NS = info.num_cores, info.num_subcores
    C = 800
    wid_flat = word_ids.reshape(N).astype(jnp.int32)
    eid_flat = entity_ids.reshape(N).astype(jnp.int32)
    out = _make_gather2(N, D, C, NC, NS)(word_table, entity_table, wid_flat, eid_flat)
    return out.reshape(B, H, 2 * D)


# speedup vs baseline: 1.8588x; 1.8588x over previous
import functools

import jax
import jax.numpy as jnp
from jax import lax
from jax.experimental import pallas as pl
from jax.experimental.pallas import tpu as pltpu
from jax.experimental.pallas import tpu_sc as plsc


def _make_gather2(N, D, C, NC, NS):
    NW = NC * NS
    per_w = N // NW
    n_chunks = per_w // C
    mesh = plsc.VectorSubcoreMesh(core_axis_name="c", subcore_axis_name="s")

    @functools.partial(
        pl.kernel,
        mesh=mesh,
        out_type=jax.ShapeDtypeStruct((N, 2 * D), jnp.float32),
        scratch_types=[
            pltpu.VMEM((C,), jnp.int32),
            pltpu.VMEM((C,), jnp.int32),
            pltpu.VMEM((C, D), jnp.float32),
            pltpu.VMEM((C, D), jnp.float32),
            pltpu.SemaphoreType.DMA,
        ],
        compiler_params=pltpu.CompilerParams(use_tc_tiling_on_sc=False),
    )
    def gather2(word_hbm, entity_hbm, wid_hbm, eid_hbm, out_hbm,
                widx_v, eidx_v, wrows_v, erows_v, sem):
        wid = lax.axis_index("s") * NC + lax.axis_index("c")
        base = wid * per_w

        def body(i, carry):
            start = base + i * C
            pltpu.sync_copy(wid_hbm.at[pl.ds(start, C)], widx_v)
            pltpu.sync_copy(eid_hbm.at[pl.ds(start, C)], eidx_v)
            cw = pltpu.async_copy(word_hbm.at[widx_v], wrows_v, sem)
            ce = pltpu.async_copy(entity_hbm.at[eidx_v], erows_v, sem)
            cw.wait()
            ce.wait()
            pltpu.sync_copy(wrows_v, out_hbm.at[pl.ds(start, C), pl.ds(0, D)])
            pltpu.sync_copy(erows_v, out_hbm.at[pl.ds(start, C), pl.ds(D, D)])
            return carry

        lax.fori_loop(0, n_chunks, body, 0, unroll=False)

    return gather2


def kernel(word_table, entity_table, word_ids, entity_ids):
    B, H = word_ids.shape
    D = word_table.shape[1]
    N = B * H
    info = plsc.get_sparse_core_info()
    NC, NS = info.num_cores, info.num_subcores
    C = 800
    wid_flat = word_ids.reshape(N).astype(jnp.int32)
    eid_flat = entity_ids.reshape(N).astype(jnp.int32)
    out = _make_gather2(N, D, C, NC, NS)(word_table, entity_table, wid_flat, eid_flat)
    return out.reshape(B, H, 2 * D)


# speedup vs baseline: 1.8607x; 1.0010x over previous
import functools

import jax
import jax.numpy as jnp
from jax import lax
from jax.experimental import pallas as pl
from jax.experimental.pallas import tpu as pltpu
from jax.experimental.pallas import tpu_sc as plsc


def _make_gather2(N, D, C, NC, NS):
    NW = NC * NS
    per_w = N // NW
    n_chunks = per_w // C
    mesh = plsc.VectorSubcoreMesh(core_axis_name="c", subcore_axis_name="s")

    @functools.partial(
        pl.kernel,
        mesh=mesh,
        out_type=jax.ShapeDtypeStruct((N, 2 * D), jnp.float32),
        scratch_types=[
            pltpu.VMEM((per_w,), jnp.int32),
            pltpu.VMEM((per_w,), jnp.int32),
            pltpu.VMEM((2, C, D), jnp.float32),
            pltpu.VMEM((2, C, D), jnp.float32),
            pltpu.SemaphoreType.DMA,
            pltpu.SemaphoreType.DMA,
            pltpu.SemaphoreType.DMA,
            pltpu.SemaphoreType.DMA,
        ],
        compiler_params=pltpu.CompilerParams(use_tc_tiling_on_sc=False),
    )
    def gather2(word_hbm, entity_hbm, wid_hbm, eid_hbm, out_hbm,
                widx_v, eidx_v, wbuf, ebuf, gsem0, gsem1, ssem0, ssem1):
        wid = lax.axis_index("s") * NC + lax.axis_index("c")
        base = wid * per_w
        gsem = (gsem0, gsem1)
        ssem = (ssem0, ssem1)

        pltpu.sync_copy(wid_hbm.at[pl.ds(base, per_w)], widx_v)
        pltpu.sync_copy(eid_hbm.at[pl.ds(base, per_w)], eidx_v)

        def issue_gather(i, p):
            return (
                pltpu.async_copy(
                    word_hbm.at[widx_v.at[pl.ds(i * C, C)]], wbuf.at[p], gsem[p]),
                pltpu.async_copy(
                    entity_hbm.at[eidx_v.at[pl.ds(i * C, C)]], ebuf.at[p], gsem[p]),
            )

        def issue_scatter(i, p):
            start = base + i * C
            return (
                pltpu.async_copy(
                    wbuf.at[p], out_hbm.at[pl.ds(start, C), pl.ds(0, D)], ssem[p]),
                pltpu.async_copy(
                    ebuf.at[p], out_hbm.at[pl.ds(start, C), pl.ds(D, D)], ssem[p]),
            )

        g = [None, None]
        sc = [None, None]
        g[0] = issue_gather(0, 0)
        for i in range(n_chunks):
            p = i % 2
            g[p][0].wait()
            g[p][1].wait()
            sc[p] = issue_scatter(i, p)
            if i + 1 < n_chunks:
                q = 1 - p
                if sc[q] is not None:
                    sc[q][0].wait()
                    sc[q][1].wait()
                g[q] = issue_gather(i + 1, q)
        for p in (0, 1):
            if sc[p] is not None:
                sc[p][0].wait()
                sc[p][1].wait()

    return gather2


def kernel(word_table, entity_table, word_ids, entity_ids):
    B, H = word_ids.shape
    D = word_table.shape[1]
    N = B * H
    info = plsc.get_sparse_core_info()
    NC, NS = info.num_cores, info.num_subcores
    C = 320
    wid_flat = word_ids.reshape(N).astype(jnp.int32)
    eid_flat = entity_ids.reshape(N).astype(jnp.int32)
    out = _make_gather2(N, D, C, NC, NS)(word_table, entity_table, wid_flat, eid_flat)
    return out.reshape(B, H, 2 * D)


# speedup vs baseline: 2.3650x; 1.2711x over previous
import functools

import jax
import jax.numpy as jnp
from jax import lax
from jax.experimental import pallas as pl
from jax.experimental.pallas import tpu as pltpu
from jax.experimental.pallas import tpu_sc as plsc


def _make_gather2(N, D, C, NC, NS):
    NW = NC * NS
    per_w = N // NW
    n_chunks = per_w // C
    mesh = plsc.VectorSubcoreMesh(core_axis_name="c", subcore_axis_name="s")

    @functools.partial(
        pl.kernel,
        mesh=mesh,
        out_type=jax.ShapeDtypeStruct((N, 2 * D), jnp.float32),
        scratch_types=[
            pltpu.VMEM((per_w,), jnp.int32),
            pltpu.VMEM((per_w,), jnp.int32),
            pltpu.VMEM((2, C, D), jnp.float32),
            pltpu.VMEM((2, C, D), jnp.float32),
            pltpu.SemaphoreType.DMA,
            pltpu.SemaphoreType.DMA,
            pltpu.SemaphoreType.DMA,
            pltpu.SemaphoreType.DMA,
        ],
        compiler_params=pltpu.CompilerParams(use_tc_tiling_on_sc=False),
    )
    def gather2(word_hbm, entity_hbm, wid_hbm, eid_hbm, out_hbm,
                widx_v, eidx_v, wbuf, ebuf, gsem0, gsem1, ssem0, ssem1):
        wid = lax.axis_index("s") * NC + lax.axis_index("c")
        base = wid * per_w
        gsem = (gsem0, gsem1)
        ssem = (ssem0, ssem1)

        pltpu.sync_copy(wid_hbm.at[pl.ds(base, per_w)], widx_v)
        pltpu.sync_copy(eid_hbm.at[pl.ds(base, per_w)], eidx_v)

        def issue_gather(i, p):
            return (
                pltpu.async_copy(
                    word_hbm.at[widx_v.at[pl.ds(i * C, C)]], wbuf.at[p], gsem[p]),
                pltpu.async_copy(
                    entity_hbm.at[eidx_v.at[pl.ds(i * C, C)]], ebuf.at[p], gsem[p]),
            )

        def issue_scatter(i, p):
            start = base + i * C
            return (
                pltpu.async_copy(
                    wbuf.at[p], out_hbm.at[pl.ds(start, C), pl.ds(0, D)], ssem[p]),
                pltpu.async_copy(
                    ebuf.at[p], out_hbm.at[pl.ds(start, C), pl.ds(D, D)], ssem[p]),
            )

        g = [None, None]
        sc = [None, None]
        g[0] = issue_gather(0, 0)
        for i in range(n_chunks):
            p = i % 2
            g[p][0].wait()
            g[p][1].wait()
            sc[p] = issue_scatter(i, p)
            if i + 1 < n_chunks:
                q = 1 - p
                if sc[q] is not None:
                    sc[q][0].wait()
                    sc[q][1].wait()
                g[q] = issue_gather(i + 1, q)
        for p in (0, 1):
            if sc[p] is not None:
                sc[p][0].wait()
                sc[p][1].wait()

    return gather2


def kernel(word_table, entity_table, word_ids, entity_ids):
    B, H = word_ids.shape
    D = word_table.shape[1]
    N = B * H
    info = plsc.get_sparse_core_info()
    NC, NS = info.num_cores, info.num_subcores
    C = 320
    wid_flat = word_ids.T.reshape(N).astype(jnp.int32)
    eid_flat = entity_ids.T.reshape(N).astype(jnp.int32)
    out = _make_gather2(N, D, C, NC, NS)(word_table, entity_table, wid_flat, eid_flat)
    return out.reshape(H, B, 2 * D).transpose(1, 0, 2)


# speedup vs baseline: 2.4915x; 1.0535x over previous
import functools

import jax
import jax.numpy as jnp
from jax import lax
from jax.experimental import pallas as pl
from jax.experimental.pallas import tpu as pltpu
from jax.experimental.pallas import tpu_sc as plsc


def _tr_body(in_ref, out_ref):
    t = in_ref[...].T
    out_ref[:, 0:t.shape[1]] = t


def _make_transpose(Dd, V, BT):
    return pl.pallas_call(
        _tr_body,
        grid=(pl.cdiv(V, BT),),
        in_specs=[pl.BlockSpec((Dd, BT), lambda i: (0, i))],
        out_specs=pl.BlockSpec((BT, 2 * Dd), lambda i: (i, 0)),
        out_shape=jax.ShapeDtypeStruct((V, 2 * Dd), jnp.float32),
    )


def _make_gather2(N, D, C, NC, NS):
    NW = NC * NS
    per_w = N // NW
    n_chunks = per_w // C
    mesh = plsc.VectorSubcoreMesh(core_axis_name="c", subcore_axis_name="s")

    @functools.partial(
        pl.kernel,
        mesh=mesh,
        out_type=jax.ShapeDtypeStruct((N, 2 * D), jnp.float32),
        scratch_types=[
            pltpu.VMEM((per_w,), jnp.int32),
            pltpu.VMEM((per_w,), jnp.int32),
            pltpu.VMEM((2, C, 2 * D), jnp.float32),
            pltpu.VMEM((2, C, 2 * D), jnp.float32),
            pltpu.SemaphoreType.DMA,
            pltpu.SemaphoreType.DMA,
            pltpu.SemaphoreType.DMA,
            pltpu.SemaphoreType.DMA,
        ],
        compiler_params=pltpu.CompilerParams(use_tc_tiling_on_sc=False),
    )
    def gather2(word_hbm, entity_hbm, wid_hbm, eid_hbm, out_hbm,
                widx_v, eidx_v, wbuf, ebuf, gsem0, gsem1, ssem0, ssem1):
        wid = lax.axis_index("s") * NC + lax.axis_index("c")
        base = wid * per_w
        gsem = (gsem0, gsem1)
        ssem = (ssem0, ssem1)

        pltpu.sync_copy(wid_hbm.at[pl.ds(base, per_w)], widx_v)
        pltpu.sync_copy(eid_hbm.at[pl.ds(base, per_w)], eidx_v)

        def issue_gather(i, p):
            return (
                pltpu.async_copy(
                    word_hbm.at[widx_v.at[pl.ds(i * C, C)]], wbuf.at[p], gsem[p]),
                pltpu.async_copy(
                    entity_hbm.at[eidx_v.at[pl.ds(i * C, C)]], ebuf.at[p], gsem[p]),
            )

        def issue_scatter(i, p):
            start = base + i * C
            return (
                pltpu.async_copy(
                    wbuf.at[p, :, pl.ds(0, D)],
                    out_hbm.at[pl.ds(start, C), pl.ds(0, D)], ssem[p]),
                pltpu.async_copy(
                    ebuf.at[p, :, pl.ds(0, D)],
                    out_hbm.at[pl.ds(start, C), pl.ds(D, D)], ssem[p]),
            )

        g = [None, None]
        sc = [None, None]
        g[0] = issue_gather(0, 0)
        for i in range(n_chunks):
            p = i % 2
            g[p][0].wait()
            g[p][1].wait()
            sc[p] = issue_scatter(i, p)
            if i + 1 < n_chunks:
                q = 1 - p
                if sc[q] is not None:
                    sc[q][0].wait()
                    sc[q][1].wait()
                g[q] = issue_gather(i + 1, q)
        for p in (0, 1):
            if sc[p] is not None:
                sc[p][0].wait()
                sc[p][1].wait()

    return gather2


def kernel(word_table, entity_table, word_ids, entity_ids):
    B, H = word_ids.shape
    D = word_table.shape[1]
    N = B * H
    info = plsc.get_sparse_core_info()
    NC, NS = info.num_cores, info.num_subcores
    C = 200
    wid_flat = word_ids.T.reshape(N).astype(jnp.int32)
    eid_flat = entity_ids.T.reshape(N).astype(jnp.int32)
    word_rm = _make_transpose(D, word_table.shape[0], 2048)(word_table.T)
    entity_rm = _make_transpose(D, entity_table.shape[0], 2048)(entity_table.T)
    out = _make_gather2(N, D, C, NC, NS)(word_rm, entity_rm, wid_flat, eid_flat)
    return out.reshape(H, B, 2 * D).transpose(1, 0, 2)


# speedup vs baseline: 3.1128x; 1.2494x over previous
import functools

import jax
import jax.numpy as jnp
from jax import lax
from jax.experimental import pallas as pl
from jax.experimental.pallas import tpu as pltpu
from jax.experimental.pallas import tpu_sc as plsc


def _tr_body(in_ref, out_ref):
    t = in_ref[...].T
    out_ref[:, 0:t.shape[1]] = t


def _make_transpose(Dd, V, BT):
    return pl.pallas_call(
        _tr_body,
        grid=(pl.cdiv(V, BT),),
        in_specs=[pl.BlockSpec((Dd, BT), lambda i: (0, i))],
        out_specs=pl.BlockSpec((BT, 2 * Dd), lambda i: (i, 0)),
        out_shape=jax.ShapeDtypeStruct((V, 2 * Dd), jnp.float32),
    )


def _make_gather2(N, D, C, NC, NS):
    NW = NC * NS
    per_w = N // NW
    n_chunks = per_w // C
    mesh = plsc.VectorSubcoreMesh(core_axis_name="c", subcore_axis_name="s")

    @functools.partial(
        pl.kernel,
        mesh=mesh,
        out_type=jax.ShapeDtypeStruct((N, 2 * D), jnp.float32),
        scratch_types=[
            pltpu.VMEM((per_w,), jnp.int32),
            pltpu.VMEM((per_w,), jnp.int32),
            pltpu.VMEM((2, C, 2 * D), jnp.float32),
            pltpu.VMEM((2, C, 2 * D), jnp.float32),
            pltpu.SemaphoreType.DMA,
            pltpu.SemaphoreType.DMA,
            pltpu.SemaphoreType.DMA,
            pltpu.SemaphoreType.DMA,
        ],
        compiler_params=pltpu.CompilerParams(use_tc_tiling_on_sc=False),
    )
    def gather2(word_hbm, entity_hbm, wid_hbm, eid_hbm, out_hbm,
                widx_v, eidx_v, wbuf, ebuf, gsem0, gsem1, ssem0, ssem1):
        wid = lax.axis_index("s") * NC + lax.axis_index("c")
        base = wid * per_w
        gsem = (gsem0, gsem1)
        ssem = (ssem0, ssem1)

        pltpu.sync_copy(wid_hbm.at[pl.ds(base, per_w)], widx_v)
        pltpu.sync_copy(eid_hbm.at[pl.ds(base, per_w)], eidx_v)

        def issue_gather(i, p):
            return (
                pltpu.async_copy(
                    word_hbm.at[widx_v.at[pl.ds(i * C, C)]], wbuf.at[p], gsem[p]),
                pltpu.async_copy(
                    entity_hbm.at[eidx_v.at[pl.ds(i * C, C)]], ebuf.at[p], gsem[p]),
            )

        def issue_scatter(i, p):
            start = base + i * C
            return (
                pltpu.async_copy(
                    wbuf.at[p, :, pl.ds(0, D)],
                    out_hbm.at[pl.ds(start, C), pl.ds(0, D)], ssem[p]),
                pltpu.async_copy(
                    ebuf.at[p, :, pl.ds(0, D)],
                    out_hbm.at[pl.ds(start, C), pl.ds(D, D)], ssem[p]),
            )

        g = [None, None]
        sc = [None, None]
        g[0] = issue_gather(0, 0)
        for i in range(n_chunks):
            p = i % 2
            g[p][0].wait()
            g[p][1].wait()
            sc[p] = issue_scatter(i, p)
            if i + 1 < n_chunks:
                q = 1 - p
                if sc[q] is not None:
                    sc[q][0].wait()
                    sc[q][1].wait()
                g[q] = issue_gather(i + 1, q)
        for p in (0, 1):
            if sc[p] is not None:
                sc[p][0].wait()
                sc[p][1].wait()

    return gather2


def kernel(word_table, entity_table, word_ids, entity_ids):
    B, H = word_ids.shape
    D = word_table.shape[1]
    N = B * H
    info = plsc.get_sparse_core_info()
    NC, NS = info.num_cores, info.num_subcores
    C = 200
    wid_flat = word_ids.T.reshape(N).astype(jnp.int32)
    eid_flat = entity_ids.T.reshape(N).astype(jnp.int32)
    word_rm = _make_transpose(D, word_table.shape[0], 4096)(word_table.T)
    entity_rm = _make_transpose(D, entity_table.shape[0], 4096)(entity_table.T)
    out = _make_gather2(N, D, C, NC, NS)(word_rm, entity_rm, wid_flat, eid_flat)
    return out.reshape(H, B, 2 * D).transpose(1, 0, 2)


# speedup vs baseline: 3.6691x; 1.1787x over previous
import functools

import jax
import jax.numpy as jnp
from jax import lax
from jax.experimental import pallas as pl
from jax.experimental.pallas import tpu as pltpu
from jax.experimental.pallas import tpu_sc as plsc


def _tr_body(in_ref, out_ref):
    t = in_ref[...].T
    out_ref[:, 0:t.shape[1]] = t


def _make_transpose(Dd, V, BT):
    return pl.pallas_call(
        _tr_body,
        grid=(pl.cdiv(V, BT),),
        in_specs=[pl.BlockSpec((Dd, BT), lambda i: (0, i))],
        out_specs=pl.BlockSpec((BT, 2 * Dd), lambda i: (i, 0)),
        out_shape=jax.ShapeDtypeStruct((V, 2 * Dd), jnp.float32),
    )


def _make_gather2(N, D, C, NC, NS):
    NW = NC * NS
    per_w = N // NW
    n_chunks = per_w // C
    mesh = plsc.VectorSubcoreMesh(core_axis_name="c", subcore_axis_name="s")

    @functools.partial(
        pl.kernel,
        mesh=mesh,
        out_type=jax.ShapeDtypeStruct((N, 2 * D), jnp.float32),
        scratch_types=[
            pltpu.VMEM((per_w,), jnp.int32),
            pltpu.VMEM((per_w,), jnp.int32),
            pltpu.VMEM((2, C, D), jnp.float32),
            pltpu.VMEM((2, C, D), jnp.float32),
            pltpu.SemaphoreType.DMA,
            pltpu.SemaphoreType.DMA,
            pltpu.SemaphoreType.DMA,
            pltpu.SemaphoreType.DMA,
        ],
        compiler_params=pltpu.CompilerParams(use_tc_tiling_on_sc=False),
    )
    def gather2(word_hbm, entity_hbm, wid_hbm, eid_hbm, out_hbm,
                widx_v, eidx_v, wbuf, ebuf, gsem0, gsem1, ssem0, ssem1):
        wid = lax.axis_index("s") * NC + lax.axis_index("c")
        base = wid * per_w
        gsem = (gsem0, gsem1)
        ssem = (ssem0, ssem1)

        pltpu.sync_copy(wid_hbm.at[pl.ds(base, per_w)], widx_v)
        pltpu.sync_copy(eid_hbm.at[pl.ds(base, per_w)], eidx_v)

        def issue_gather(i, p):
            return (
                pltpu.async_copy(
                    word_hbm.at[widx_v.at[pl.ds(i * C, C)]], wbuf.at[p], gsem[p]),
                pltpu.async_copy(
                    entity_hbm.at[eidx_v.at[pl.ds(i * C, C)]], ebuf.at[p], gsem[p]),
            )

        def issue_scatter(i, p):
            start = base + i * C
            return (
                pltpu.async_copy(
                    wbuf.at[p], out_hbm.at[pl.ds(start, C), pl.ds(0, D)], ssem[p]),
                pltpu.async_copy(
                    ebuf.at[p], out_hbm.at[pl.ds(start, C), pl.ds(D, D)], ssem[p]),
            )

        g = [None, None]
        sc = [None, None]
        g[0] = issue_gather(0, 0)
        for i in range(n_chunks):
            p = i % 2
            g[p][0].wait()
            g[p][1].wait()
            sc[p] = issue_scatter(i, p)
            if i + 1 < n_chunks:
                q = 1 - p
                if sc[q] is not None:
                    sc[q][0].wait()
                    sc[q][1].wait()
                g[q] = issue_gather(i + 1, q)
        for p in (0, 1):
            if sc[p] is not None:
                sc[p][0].wait()
                sc[p][1].wait()

    return gather2


def kernel(word_table, entity_table, word_ids, entity_ids):
    B, H = word_ids.shape
    D = word_table.shape[1]
    N = B * H
    info = plsc.get_sparse_core_info()
    NC, NS = info.num_cores, info.num_subcores
    C = 320
    wid_flat = word_ids.T.reshape(N).astype(jnp.int32) * 2
    eid_flat = entity_ids.T.reshape(N).astype(jnp.int32) * 2
    word_rm = _make_transpose(D, word_table.shape[0], 4096)(word_table.T)
    entity_rm = _make_transpose(D, entity_table.shape[0], 4096)(entity_table.T)
    word_rm = word_rm.reshape(2 * word_table.shape[0], D)
    entity_rm = entity_rm.reshape(2 * entity_table.shape[0], D)
    out = _make_gather2(N, D, C, NC, NS)(word_rm, entity_rm, wid_flat, eid_flat)
    return out.reshape(H, B, 2 * D).transpose(1, 0, 2)


# speedup vs baseline: 4.4467x; 1.2119x over previous
import functools

import jax
import jax.numpy as jnp
from jax import lax
from jax.experimental import pallas as pl
from jax.experimental.pallas import tpu as pltpu
from jax.experimental.pallas import tpu_sc as plsc


def _tr_body(lo_ref, hi_ref, out_ref):
    out_ref[...] = jnp.concatenate([lo_ref[...].T, hi_ref[...].T], axis=1)


def _make_transpose(Dd, V, BT):
    S = BT * (V // (2 * BT))
    Vh = V - S
    nb_lo = S // BT
    return pl.pallas_call(
        _tr_body,
        grid=(pl.cdiv(Vh, BT),),
        in_specs=[
            pl.BlockSpec((Dd, BT), lambda i: (0, i)),
            pl.BlockSpec((Dd, BT), lambda i: (0, i + nb_lo)),
        ],
        out_specs=pl.BlockSpec((BT, 2 * Dd), lambda i: (i, 0)),
        out_shape=jax.ShapeDtypeStruct((Vh, 2 * Dd), jnp.float32),
    ), S, Vh


def _make_gather2(N, D, C, NC, NS):
    NW = NC * NS
    per_w = N // NW
    n_chunks = per_w // C
    mesh = plsc.VectorSubcoreMesh(core_axis_name="c", subcore_axis_name="s")

    @functools.partial(
        pl.kernel,
        mesh=mesh,
        out_type=jax.ShapeDtypeStruct((N, 2 * D), jnp.float32),
        scratch_types=[
            pltpu.VMEM((per_w,), jnp.int32),
            pltpu.VMEM((per_w,), jnp.int32),
            pltpu.VMEM((2, C, D), jnp.float32),
            pltpu.VMEM((2, C, D), jnp.float32),
            pltpu.SemaphoreType.DMA,
            pltpu.SemaphoreType.DMA,
            pltpu.SemaphoreType.DMA,
            pltpu.SemaphoreType.DMA,
        ],
        compiler_params=pltpu.CompilerParams(use_tc_tiling_on_sc=False),
    )
    def gather2(word_hbm, entity_hbm, wid_hbm, eid_hbm, out_hbm,
                widx_v, eidx_v, wbuf, ebuf, gsem0, gsem1, ssem0, ssem1):
        wid = lax.axis_index("s") * NC + lax.axis_index("c")
        base = wid * per_w
        gsem = (gsem0, gsem1)
        ssem = (ssem0, ssem1)

        pltpu.sync_copy(wid_hbm.at[pl.ds(base, per_w)], widx_v)
        pltpu.sync_copy(eid_hbm.at[pl.ds(base, per_w)], eidx_v)

        def issue_gather(i, p):
            return (
                pltpu.async_copy(
                    word_hbm.at[widx_v.at[pl.ds(i * C, C)]], wbuf.at[p], gsem[p]),
                pltpu.async_copy(
                    entity_hbm.at[eidx_v.at[pl.ds(i * C, C)]], ebuf.at[p], gsem[p]),
            )

        def issue_scatter(i, p):
            start = base + i * C
            return (
                pltpu.async_copy(
                    wbuf.at[p], out_hbm.at[pl.ds(start, C), pl.ds(0, D)], ssem[p]),
                pltpu.async_copy(
                    ebuf.at[p], out_hbm.at[pl.ds(start, C), pl.ds(D, D)], ssem[p]),
            )

        g = [None, None]
        sc = [None, None]
        g[0] = issue_gather(0, 0)
        for i in range(n_chunks):
            p = i % 2
            g[p][0].wait()
            g[p][1].wait()
            sc[p] = issue_scatter(i, p)
            if i + 1 < n_chunks:
                q = 1 - p
                if sc[q] is not None:
                    sc[q][0].wait()
                    sc[q][1].wait()
                g[q] = issue_gather(i + 1, q)
        for p in (0, 1):
            if sc[p] is not None:
                sc[p][0].wait()
                sc[p][1].wait()

    return gather2


def kernel(word_table, entity_table, word_ids, entity_ids):
    B, H = word_ids.shape
    D = word_table.shape[1]
    N = B * H
    info = plsc.get_sparse_core_info()
    NC, NS = info.num_cores, info.num_subcores
    C = 320
    VW = word_table.shape[0]
    VE = entity_table.shape[0]
    wT = word_table.T
    eT = entity_table.T
    tr_w, SW, VhW = _make_transpose(D, VW, 4096)
    tr_e, SE, VhE = _make_transpose(D, VE, 2048)
    word_rm = tr_w(wT, wT).reshape(2 * VhW, D)
    entity_rm = tr_e(eT, eT).reshape(2 * VhE, D)
    wv = word_ids.T.reshape(N).astype(jnp.int32)
    ev = entity_ids.T.reshape(N).astype(jnp.int32)
    wid_flat = jnp.where(wv < SW, 2 * wv, 2 * (wv - SW) + 1)
    eid_flat = jnp.where(ev < SE, 2 * ev, 2 * (ev - SE) + 1)
    out = _make_gather2(N, D, C, NC, NS)(word_rm, entity_rm, wid_flat, eid_flat)
    return out.reshape(H, B, 2 * D).transpose(1, 0, 2)


# speedup vs baseline: 4.8416x; 1.0888x over previous
import functools

import jax
import jax.numpy as jnp
from jax import lax
from jax.experimental import pallas as pl
from jax.experimental.pallas import tpu as pltpu
from jax.experimental.pallas import tpu_sc as plsc


def _tr_body(lo_ref, hi_ref, out_ref):
    d = lo_ref.shape[0]
    out_ref[:, 0:d] = lo_ref[...].T
    out_ref[:, d:2 * d] = hi_ref[...].T


def _make_transpose(Dd, V, BT):
    S = BT * (V // (2 * BT))
    Vh = V - S
    nb_lo = S // BT
    return pl.pallas_call(
        _tr_body,
        grid=(pl.cdiv(Vh, BT),),
        in_specs=[
            pl.BlockSpec((Dd, BT), lambda i: (0, i)),
            pl.BlockSpec((Dd, BT), lambda i: (0, i + nb_lo)),
        ],
        out_specs=pl.BlockSpec((BT, 2 * Dd), lambda i: (i, 0)),
        out_shape=jax.ShapeDtypeStruct((Vh, 2 * Dd), jnp.float32),
    ), S, Vh


def _make_gather2(N, D, C, NC, NS):
    NW = NC * NS
    per_w = N // NW
    n_chunks = per_w // C
    mesh = plsc.VectorSubcoreMesh(core_axis_name="c", subcore_axis_name="s")

    @functools.partial(
        pl.kernel,
        mesh=mesh,
        out_type=jax.ShapeDtypeStruct((N, 2 * D), jnp.float32),
        scratch_types=[
            pltpu.VMEM((per_w,), jnp.int32),
            pltpu.VMEM((per_w,), jnp.int32),
            pltpu.VMEM((2, C, D), jnp.float32),
            pltpu.VMEM((2, C, D), jnp.float32),
            pltpu.SemaphoreType.DMA,
            pltpu.SemaphoreType.DMA,
            pltpu.SemaphoreType.DMA,
            pltpu.SemaphoreType.DMA,
        ],
        compiler_params=pltpu.CompilerParams(use_tc_tiling_on_sc=False),
    )
    def gather2(word_hbm, entity_hbm, wid_hbm, eid_hbm, out_hbm,
                widx_v, eidx_v, wbuf, ebuf, gsem0, gsem1, ssem0, ssem1):
        wid = lax.axis_index("s") * NC + lax.axis_index("c")
        base = wid * per_w
        gsem = (gsem0, gsem1)
        ssem = (ssem0, ssem1)

        pltpu.sync_copy(wid_hbm.at[pl.ds(base, per_w)], widx_v)
        pltpu.sync_copy(eid_hbm.at[pl.ds(base, per_w)], eidx_v)

        def issue_gather(i, p):
            return (
                pltpu.async_copy(
                    word_hbm.at[widx_v.at[pl.ds(i * C, C)]], wbuf.at[p], gsem[p]),
                pltpu.async_copy(
                    entity_hbm.at[eidx_v.at[pl.ds(i * C, C)]], ebuf.at[p], gsem[p]),
            )

        def issue_scatter(i, p):
            start = base + i * C
            return (
                pltpu.async_copy(
                    wbuf.at[p], out_hbm.at[pl.ds(start, C), pl.ds(0, D)], ssem[p]),
                pltpu.async_copy(
                    ebuf.at[p], out_hbm.at[pl.ds(start, C), pl.ds(D, D)], ssem[p]),
            )

        g = [None, None]
        sc = [None, None]
        g[0] = issue_gather(0, 0)
        for i in range(n_chunks):
            p = i % 2
            g[p][0].wait()
            g[p][1].wait()
            sc[p] = issue_scatter(i, p)
            if i + 1 < n_chunks:
                q = 1 - p
                if sc[q] is not None:
                    sc[q][0].wait()
                    sc[q][1].wait()
                g[q] = issue_gather(i + 1, q)
        for p in (0, 1):
            if sc[p] is not None:
                sc[p][0].wait()
                sc[p][1].wait()

    return gather2


def kernel(word_table, entity_table, word_ids, entity_ids):
    B, H = word_ids.shape
    D = word_table.shape[1]
    N = B * H
    info = plsc.get_sparse_core_info()
    NC, NS = info.num_cores, info.num_subcores
    C = 320
    VW = word_table.shape[0]
    VE = entity_table.shape[0]
    wT = word_table.T
    eT = entity_table.T
    tr_w, SW, VhW = _make_transpose(D, VW, 8192)
    tr_e, SE, VhE = _make_transpose(D, VE, 2048)
    word_rm = tr_w(wT, wT).reshape(2 * VhW, D)
    entity_rm = tr_e(eT, eT).reshape(2 * VhE, D)
    wv = word_ids.T.reshape(N).astype(jnp.int32)
    ev = entity_ids.T.reshape(N).astype(jnp.int32)
    wid_flat = jnp.where(wv < SW, 2 * wv, 2 * (wv - SW) + 1)
    eid_flat = jnp.where(ev < SE, 2 * ev, 2 * (ev - SE) + 1)
    out = _make_gather2(N, D, C, NC, NS)(word_rm, entity_rm, wid_flat, eid_flat)
    return out.reshape(H, B, 2 * D).transpose(1, 0, 2)


# speedup vs baseline: 5.0659x; 1.0463x over previous
import functools

import jax
import jax.numpy as jnp
from jax import lax
from jax.experimental import pallas as pl
from jax.experimental.pallas import tpu as pltpu
from jax.experimental.pallas import tpu_sc as plsc


def _tr_body(lo_ref, hi_ref, out_ref):
    d = lo_ref.shape[0]
    out_ref[:, 0:d] = lo_ref[...].T
    out_ref[:, d:2 * d] = hi_ref[...].T


def _make_transpose(Dd, V, BT):
    S = BT * (V // (2 * BT))
    Vh = V - S
    nb_lo = S // BT
    return pl.pallas_call(
        _tr_body,
        grid=(pl.cdiv(Vh, BT),),
        in_specs=[
            pl.BlockSpec((Dd, BT), lambda i: (0, i)),
            pl.BlockSpec((Dd, BT), lambda i: (0, i + nb_lo)),
        ],
        out_specs=pl.BlockSpec((BT, 2 * Dd), lambda i: (i, 0)),
        out_shape=jax.ShapeDtypeStruct((Vh, 2 * Dd), jnp.float32),
    ), S, Vh


def _make_gather2(N, D, C, NC, NS):
    NW = NC * NS
    per_w = N // NW
    n_chunks = per_w // C
    mesh = plsc.VectorSubcoreMesh(core_axis_name="c", subcore_axis_name="s")

    @functools.partial(
        pl.kernel,
        mesh=mesh,
        out_type=jax.ShapeDtypeStruct((N, 2 * D), jnp.float32),
        scratch_types=[
            pltpu.VMEM((per_w,), jnp.int32),
            pltpu.VMEM((per_w,), jnp.int32),
            pltpu.VMEM((2, C, D), jnp.float32),
            pltpu.VMEM((2, C, D), jnp.float32),
            pltpu.SemaphoreType.DMA,
            pltpu.SemaphoreType.DMA,
            pltpu.SemaphoreType.DMA,
            pltpu.SemaphoreType.DMA,
        ],
        compiler_params=pltpu.CompilerParams(use_tc_tiling_on_sc=False),
    )
    def gather2(word_hbm, entity_hbm, wid_hbm, eid_hbm, out_hbm,
                widx_v, eidx_v, wbuf, ebuf, gsem0, gsem1, ssem0, ssem1):
        wid = lax.axis_index("s") * NC + lax.axis_index("c")
        base = wid * per_w
        gsem = (gsem0, gsem1)
        ssem = (ssem0, ssem1)

        pltpu.sync_copy(wid_hbm.at[pl.ds(base, per_w)], widx_v)
        pltpu.sync_copy(eid_hbm.at[pl.ds(base, per_w)], eidx_v)

        def issue_gather(i, p):
            return (
                pltpu.async_copy(
                    word_hbm.at[widx_v.at[pl.ds(i * C, C)]], wbuf.at[p], gsem[p]),
                pltpu.async_copy(
                    entity_hbm.at[eidx_v.at[pl.ds(i * C, C)]], ebuf.at[p], gsem[p]),
            )

        def issue_scatter(i, p):
            start = base + i * C
            return (
                pltpu.async_copy(
                    wbuf.at[p], out_hbm.at[pl.ds(start, C), pl.ds(0, D)], ssem[p]),
                pltpu.async_copy(
                    ebuf.at[p], out_hbm.at[pl.ds(start, C), pl.ds(D, D)], ssem[p]),
            )

        g = [None, None]
        sc = [None, None]
        g[0] = issue_gather(0, 0)
        for i in range(n_chunks):
            p = i % 2
            g[p][0].wait()
            g[p][1].wait()
            sc[p] = issue_scatter(i, p)
            if i + 1 < n_chunks:
                q = 1 - p
                if sc[q] is not None:
                    sc[q][0].wait()
                    sc[q][1].wait()
                g[q] = issue_gather(i + 1, q)
        for p in (0, 1):
            if sc[p] is not None:
                sc[p][0].wait()
                sc[p][1].wait()

    return gather2


def kernel(word_table, entity_table, word_ids, entity_ids):
    B, H = word_ids.shape
    D = word_table.shape[1]
    N = B * H
    info = plsc.get_sparse_core_info()
    NC, NS = info.num_cores, info.num_subcores
    C = 320
    VW = word_table.shape[0]
    VE = entity_table.shape[0]
    wT = word_table.T
    eT = entity_table.T
    tr_w, SW, VhW = _make_transpose(D, VW, 16384)
    tr_e, SE, VhE = _make_transpose(D, VE, 4096)
    word_rm = tr_w(wT, wT).reshape(2 * VhW, D)
    entity_rm = tr_e(eT, eT).reshape(2 * VhE, D)
    wv = word_ids.T.reshape(N).astype(jnp.int32)
    ev = entity_ids.T.reshape(N).astype(jnp.int32)
    wid_flat = jnp.where(wv < SW, 2 * wv, 2 * (wv - SW) + 1)
    eid_flat = jnp.where(ev < SE, 2 * ev, 2 * (ev - SE) + 1)
    out = _make_gather2(N, D, C, NC, NS)(word_rm, entity_rm, wid_flat, eid_flat)
    return out.reshape(H, B, 2 * D).transpose(1, 0, 2)


# speedup vs baseline: 5.0918x; 1.0051x over previous
import functools

import jax
import jax.numpy as jnp
from jax import lax
from jax.experimental import pallas as pl
from jax.experimental.pallas import tpu as pltpu
from jax.experimental.pallas import tpu_sc as plsc


def _tr_body(lo_ref, hi_ref, out_ref):
    d = lo_ref.shape[0]
    out_ref[:, 0:d] = lo_ref[...].T
    out_ref[:, d:2 * d] = hi_ref[...].T


def _make_transpose(Dd, V, BT):
    S = BT * (V // (2 * BT))
    Vh = V - S
    nb_lo = S // BT
    return pl.pallas_call(
        _tr_body,
        grid=(pl.cdiv(Vh, BT),),
        in_specs=[
            pl.BlockSpec((Dd, BT), lambda i: (0, i)),
            pl.BlockSpec((Dd, BT), lambda i: (0, i + nb_lo)),
        ],
        out_specs=pl.BlockSpec((BT, 2 * Dd), lambda i: (i, 0)),
        out_shape=jax.ShapeDtypeStruct((Vh, 2 * Dd), jnp.float32),
    ), S, Vh


def _make_gather2(N, D, C, NC, NS):
    NW = NC * NS
    per_w = N // NW
    n_chunks = per_w // C
    mesh = plsc.VectorSubcoreMesh(core_axis_name="c", subcore_axis_name="s")

    @functools.partial(
        pl.kernel,
        mesh=mesh,
        out_type=jax.ShapeDtypeStruct((N, 2 * D), jnp.float32),
        scratch_types=[
            pltpu.VMEM((per_w,), jnp.int32),
            pltpu.VMEM((per_w,), jnp.int32),
            pltpu.VMEM((2, C, D), jnp.float32),
            pltpu.VMEM((2, C, D), jnp.float32),
            pltpu.SemaphoreType.DMA,
            pltpu.SemaphoreType.DMA,
            pltpu.SemaphoreType.DMA,
            pltpu.SemaphoreType.DMA,
        ],
        compiler_params=pltpu.CompilerParams(use_tc_tiling_on_sc=False),
    )
    def gather2(word_hbm, entity_hbm, wid_hbm, eid_hbm, out_hbm,
                widx_v, eidx_v, wbuf, ebuf, gsem0, gsem1, ssem0, ssem1):
        wid = lax.axis_index("s") * NC + lax.axis_index("c")
        base = wid * per_w
        gsem = (gsem0, gsem1)
        ssem = (ssem0, ssem1)

        pltpu.sync_copy(wid_hbm.at[pl.ds(base, per_w)], widx_v)
        pltpu.sync_copy(eid_hbm.at[pl.ds(base, per_w)], eidx_v)

        def issue_gather(i, p):
            return (
                pltpu.async_copy(
                    word_hbm.at[widx_v.at[pl.ds(i * C, C)]], wbuf.at[p], gsem[p]),
                pltpu.async_copy(
                    entity_hbm.at[eidx_v.at[pl.ds(i * C, C)]], ebuf.at[p], gsem[p]),
            )

        def issue_scatter(i, p):
            start = base + i * C
            return (
                pltpu.async_copy(
                    wbuf.at[p], out_hbm.at[pl.ds(start, C), pl.ds(0, D)], ssem[p]),
                pltpu.async_copy(
                    ebuf.at[p], out_hbm.at[pl.ds(start, C), pl.ds(D, D)], ssem[p]),
            )

        g = [None, None]
        sc = [None, None]
        g[0] = issue_gather(0, 0)
        for i in range(n_chunks):
            p = i % 2
            g[p][0].wait()
            g[p][1].wait()
            sc[p] = issue_scatter(i, p)
            if i + 1 < n_chunks:
                q = 1 - p
                if sc[q] is not None:
                    sc[q][0].wait()
                    sc[q][1].wait()
                g[q] = issue_gather(i + 1, q)
        for p in (0, 1):
            if sc[p] is not None:
                sc[p][0].wait()
                sc[p][1].wait()

    return gather2


def kernel(word_table, entity_table, word_ids, entity_ids):
    B, H = word_ids.shape
    D = word_table.shape[1]
    N = B * H
    info = plsc.get_sparse_core_info()
    NC, NS = info.num_cores, info.num_subcores
    C = 400
    VW = word_table.shape[0]
    VE = entity_table.shape[0]
    wT = word_table.T
    eT = entity_table.T
    tr_w, SW, VhW = _make_transpose(D, VW, 16384)
    tr_e, SE, VhE = _make_transpose(D, VE, 8192)
    word_rm = tr_w(wT, wT).reshape(2 * VhW, D)
    entity_rm = tr_e(eT, eT).reshape(2 * VhE, D)
    wv = word_ids.T.reshape(N).astype(jnp.int32)
    ev = entity_ids.T.reshape(N).astype(jnp.int32)
    wid_flat = jnp.where(wv < SW, 2 * wv, 2 * (wv - SW) + 1)
    eid_flat = jnp.where(ev < SE, 2 * ev, 2 * (ev - SE) + 1)
    out = _make_gather2(N, D, C, NC, NS)(word_rm, entity_rm, wid_flat, eid_flat)
    return out.reshape(H, B, 2 * D).transpose(1, 0, 2)


# speedup vs baseline: 5.1051x; 1.0026x over previous
import functools

import jax
import jax.numpy as jnp
from jax import lax
from jax.experimental import pallas as pl
from jax.experimental.pallas import tpu as pltpu
from jax.experimental.pallas import tpu_sc as plsc


def _tr_body(lo_ref, hi_ref, out_ref):
    d = lo_ref.shape[0]
    out_ref[:, 0:d] = lo_ref[...].T
    out_ref[:, d:2 * d] = hi_ref[...].T


def _make_transpose(Dd, V, BT):
    S = BT * (V // (2 * BT))
    Vh = V - S
    nb_lo = S // BT
    return pl.pallas_call(
        _tr_body,
        grid=(pl.cdiv(Vh, BT),),
        in_specs=[
            pl.BlockSpec((Dd, BT), lambda i: (0, i)),
            pl.BlockSpec((Dd, BT), lambda i: (0, i + nb_lo)),
        ],
        out_specs=pl.BlockSpec((BT, 2 * Dd), lambda i: (i, 0)),
        out_shape=jax.ShapeDtypeStruct((Vh, 2 * Dd), jnp.float32),
        compiler_params=pltpu.CompilerParams(vmem_limit_bytes=112 * 1024 * 1024),
    ), S, Vh


def _make_gather2(N, D, C, NC, NS):
    NW = NC * NS
    per_w = N // NW
    n_chunks = per_w // C
    mesh = plsc.VectorSubcoreMesh(core_axis_name="c", subcore_axis_name="s")

    @functools.partial(
        pl.kernel,
        mesh=mesh,
        out_type=jax.ShapeDtypeStruct((N, 2 * D), jnp.float32),
        scratch_types=[
            pltpu.VMEM((per_w,), jnp.int32),
            pltpu.VMEM((per_w,), jnp.int32),
            pltpu.VMEM((2, C, D), jnp.float32),
            pltpu.VMEM((2, C, D), jnp.float32),
            pltpu.SemaphoreType.DMA,
            pltpu.SemaphoreType.DMA,
            pltpu.SemaphoreType.DMA,
            pltpu.SemaphoreType.DMA,
        ],
        compiler_params=pltpu.CompilerParams(use_tc_tiling_on_sc=False),
    )
    def gather2(word_hbm, entity_hbm, wid_hbm, eid_hbm, out_hbm,
                widx_v, eidx_v, wbuf, ebuf, gsem0, gsem1, ssem0, ssem1):
        wid = lax.axis_index("s") * NC + lax.axis_index("c")
        base = wid * per_w
        gsem = (gsem0, gsem1)
        ssem = (ssem0, ssem1)

        pltpu.sync_copy(wid_hbm.at[pl.ds(base, per_w)], widx_v)
        pltpu.sync_copy(eid_hbm.at[pl.ds(base, per_w)], eidx_v)

        def issue_gather(i, p):
            return (
                pltpu.async_copy(
                    word_hbm.at[widx_v.at[pl.ds(i * C, C)]], wbuf.at[p], gsem[p]),
                pltpu.async_copy(
                    entity_hbm.at[eidx_v.at[pl.ds(i * C, C)]], ebuf.at[p], gsem[p]),
            )

        def issue_scatter(i, p):
            start = base + i * C
            return (
                pltpu.async_copy(
                    wbuf.at[p], out_hbm.at[pl.ds(start, C), pl.ds(0, D)], ssem[p]),
                pltpu.async_copy(
                    ebuf.at[p], out_hbm.at[pl.ds(start, C), pl.ds(D, D)], ssem[p]),
            )

        g = [None, None]
        sc = [None, None]
        g[0] = issue_gather(0, 0)
        for i in range(n_chunks):
            p = i % 2
            g[p][0].wait()
            g[p][1].wait()
            sc[p] = issue_scatter(i, p)
            if i + 1 < n_chunks:
                q = 1 - p
                if sc[q] is not None:
                    sc[q][0].wait()
                    sc[q][1].wait()
                g[q] = issue_gather(i + 1, q)
        for p in (0, 1):
            if sc[p] is not None:
                sc[p][0].wait()
                sc[p][1].wait()

    return gather2


def kernel(word_table, entity_table, word_ids, entity_ids):
    B, H = word_ids.shape
    D = word_table.shape[1]
    N = B * H
    info = plsc.get_sparse_core_info()
    NC, NS = info.num_cores, info.num_subcores
    C = 400
    VW = word_table.shape[0]
    VE = entity_table.shape[0]
    wT = word_table.T
    eT = entity_table.T
    tr_w, SW, VhW = _make_transpose(D, VW, 24576)
    tr_e, SE, VhE = _make_transpose(D, VE, 8192)
    word_rm = tr_w(wT, wT).reshape(2 * VhW, D)
    entity_rm = tr_e(eT, eT).reshape(2 * VhE, D)
    wv = word_ids.T.reshape(N).astype(jnp.int32)
    ev = entity_ids.T.reshape(N).astype(jnp.int32)
    wid_flat = jnp.where(wv < SW, 2 * wv, 2 * (wv - SW) + 1)
    eid_flat = jnp.where(ev < SE, 2 * ev, 2 * (ev - SE) + 1)
    out = _make_gather2(N, D, C, NC, NS)(word_rm, entity_rm, wid_flat, eid_flat)
    return out.reshape(H, B, 2 * D).transpose(1, 0, 2)
